# baseline (device time: 256660 ns/iter reference)
import jax
import jax.numpy as jnp
from jax import lax
from jax.experimental import pallas as pl
from jax.experimental.pallas import tpu as pltpu

N_DEV = 16
N_EXP = 32
TOK = 512
D = 256
H = 512


def kernel(x, router_W, route_idx, expert_W):
    def body(x_ref, rw_ref, idx_ref, ew_ref, out_ref,
             comm_ref, send_sems, recv_sems, credit_sem):
        my = lax.axis_index("i")
        left = lax.rem(my + N_DEV - 1, N_DEV)
        right = lax.rem(my + 1, N_DEV)

        barrier_sem = pltpu.get_barrier_semaphore()
        for nbr in (left, right):
            pl.semaphore_signal(barrier_sem, inc=1, device_id=(nbr,),
                                device_id_type=pl.DeviceIdType.MESH)
        pl.semaphore_wait(barrier_sem, 2)

        pl.semaphore_signal(credit_sem, inc=1, device_id=(left,),
                            device_id_type=pl.DeviceIdType.MESH)

        xv = x_ref[:, :]
        scores = jnp.dot(xv, rw_ref[:, :], preferred_element_type=jnp.float32)
        p = jnp.exp(scores - jnp.max(scores, axis=-1, keepdims=True))
        p = p / jnp.sum(p, axis=-1, keepdims=True)
        idx = idx_ref[:, :]
        idx0 = idx[:, 0:1]
        idx1 = idx[:, 1:2]
        eids = lax.broadcasted_iota(jnp.int32, (TOK, N_EXP), 1)
        p0 = jnp.sum(jnp.where(eids == idx0, p, 0.0), axis=-1, keepdims=True)
        p1 = jnp.sum(jnp.where(eids == idx1, p, 0.0), axis=-1, keepdims=True)
        w0 = p0 / (p0 + p1)
        w1 = p1 / (p0 + p1)

        def contrib(w_pair, origin):
            e0 = 2 * origin
            e1 = e0 + 1
            g0 = jnp.where(idx0 == e0, w0, 0.0) + jnp.where(idx1 == e0, w1, 0.0)
            g1 = jnp.where(idx0 == e1, w0, 0.0) + jnp.where(idx1 == e1, w1, 0.0)
            xg = jnp.concatenate([xv * g0, xv * g1], axis=1)
            return jnp.dot(xg, w_pair.reshape(2 * D, H),
                           preferred_element_type=jnp.float32)

        for h in range(N_DEV - 1):
            s = h % 2
            r = (h + 1) % 2
            pl.semaphore_wait(credit_sem, 1)
            rdma = pltpu.make_async_remote_copy(
                src_ref=ew_ref if h == 0 else comm_ref.at[s],
                dst_ref=comm_ref.at[r],
                send_sem=send_sems.at[s],
                recv_sem=recv_sems.at[r],
                device_id=(right,),
                device_id_type=pl.DeviceIdType.MESH,
            )
            rdma.start()
            if h == 0:
                out_ref[:, :] = contrib(ew_ref[:, :, :], my)
            else:
                origin = lax.rem(my - h + N_DEV, N_DEV)
                out_ref[:, :] += contrib(comm_ref[s], origin)
            rdma.wait()
            if h < N_DEV - 2:
                pl.semaphore_signal(credit_sem, inc=1, device_id=(left,),
                                    device_id_type=pl.DeviceIdType.MESH)

        out_ref[:, :] += contrib(comm_ref[1], lax.rem(my + 1, N_DEV))

    return pl.pallas_call(
        body,
        out_shape=jax.ShapeDtypeStruct((TOK, H), jnp.float32),
        in_specs=[pl.BlockSpec(memory_space=pltpu.VMEM)] * 4,
        out_specs=pl.BlockSpec(memory_space=pltpu.VMEM),
        scratch_shapes=[
            pltpu.VMEM((2, 2, D, H), jnp.float32),
            pltpu.SemaphoreType.DMA((2,)),
            pltpu.SemaphoreType.DMA((2,)),
            pltpu.SemaphoreType.REGULAR,
        ],
        compiler_params=pltpu.CompilerParams(collective_id=0),
    )(x, router_W, route_idx, expert_W)


# device time: 81925 ns/iter; 3.1329x vs baseline; 3.1329x over previous
import jax
import jax.numpy as jnp
from jax import lax
from jax.experimental import pallas as pl
from jax.experimental.pallas import tpu as pltpu

N_DEV = 16
N_EXP = 32
TOK = 512
D = 256
H = 512
S = 4
R_HOPS = 8
L_HOPS = 7


def kernel(x, router_W, route_idx, expert_W):
    def body(x_ref, rw_ref, idx_ref, ew_ref, out_ref,
             ew_bf_ref, comm_r_ref, comm_l_ref,
             send_sems_r, recv_sems_r, send_sems_l, recv_sems_l,
             credit_r, credit_l):
        my = lax.axis_index("i")
        left = lax.rem(my + N_DEV - 1, N_DEV)
        right = lax.rem(my + 1, N_DEV)

        barrier_sem = pltpu.get_barrier_semaphore()
        for nbr in (left, right):
            pl.semaphore_signal(barrier_sem, inc=1, device_id=(nbr,),
                                device_id_type=pl.DeviceIdType.MESH)
        pl.semaphore_wait(barrier_sem, 2)

        pl.semaphore_signal(credit_r, inc=S, device_id=(left,),
                            device_id_type=pl.DeviceIdType.MESH)
        pl.semaphore_signal(credit_l, inc=S, device_id=(right,),
                            device_id_type=pl.DeviceIdType.MESH)

        ew_bf_ref[:, :, :] = ew_ref[:, :, :].astype(jnp.bfloat16)

        xv = x_ref[:, :]
        scores = jnp.dot(xv, rw_ref[:, :], preferred_element_type=jnp.float32)
        p = jnp.exp(scores - jnp.max(scores, axis=-1, keepdims=True))
        p = p / jnp.sum(p, axis=-1, keepdims=True)
        idx = idx_ref[:, :]
        idx0 = idx[:, 0:1]
        idx1 = idx[:, 1:2]
        eids = lax.broadcasted_iota(jnp.int32, (TOK, N_EXP), 1)
        p0 = jnp.sum(jnp.where(eids == idx0, p, 0.0), axis=-1, keepdims=True)
        p1 = jnp.sum(jnp.where(eids == idx1, p, 0.0), axis=-1, keepdims=True)
        w0 = p0 / (p0 + p1)
        w1 = p1 / (p0 + p1)

        def contrib(w_pair_bf, origin):
            e0 = 2 * origin
            e1 = e0 + 1
            g0 = jnp.where(idx0 == e0, w0, 0.0) + jnp.where(idx1 == e0, w1, 0.0)
            g1 = jnp.where(idx0 == e1, w0, 0.0) + jnp.where(idx1 == e1, w1, 0.0)
            xg = jnp.concatenate([xv * g0, xv * g1], axis=1)
            return jnp.dot(xg.astype(jnp.bfloat16), w_pair_bf.reshape(2 * D, H),
                           preferred_element_type=jnp.float32)

        def ring_rdma(comm_ref, send_sems, recv_sems, h, nbr):
            return pltpu.make_async_remote_copy(
                src_ref=ew_bf_ref if h == 0 else comm_ref.at[(h - 1) % S],
                dst_ref=comm_ref.at[h % S],
                send_sem=send_sems.at[h % S],
                recv_sem=recv_sems.at[h % S],
                device_id=(nbr,),
                device_id_type=pl.DeviceIdType.MESH,
            )

        for h in range(R_HOPS):
            pl.semaphore_wait(credit_r, 1)
            rdma_r = ring_rdma(comm_r_ref, send_sems_r, recv_sems_r, h, right)
            rdma_r.start()
            if h < L_HOPS:
                pl.semaphore_wait(credit_l, 1)
                rdma_l = ring_rdma(comm_l_ref, send_sems_l, recv_sems_l, h, left)
                rdma_l.start()

            if h == 0:
                out_ref[:, :] = contrib(ew_bf_ref[:, :, :], my)
            else:
                out_ref[:, :] += contrib(comm_r_ref[(h - 1) % S],
                                         lax.rem(my - h + N_DEV, N_DEV))
                out_ref[:, :] += contrib(comm_l_ref[(h - 1) % S],
                                         lax.rem(my + h, N_DEV))

            rdma_r.wait()
            if h < L_HOPS:
                rdma_l.wait()
            if 1 <= h <= R_HOPS - S:
                pl.semaphore_signal(credit_r, inc=1, device_id=(left,),
                                    device_id_type=pl.DeviceIdType.MESH)
            if 1 <= h <= L_HOPS - S:
                pl.semaphore_signal(credit_l, inc=1, device_id=(right,),
                                    device_id_type=pl.DeviceIdType.MESH)

        out_ref[:, :] += contrib(comm_r_ref[(R_HOPS - 1) % S],
                                 lax.rem(my - R_HOPS + N_DEV, N_DEV))

    return pl.pallas_call(
        body,
        out_shape=jax.ShapeDtypeStruct((TOK, H), jnp.float32),
        in_specs=[pl.BlockSpec(memory_space=pltpu.VMEM)] * 4,
        out_specs=pl.BlockSpec(memory_space=pltpu.VMEM),
        scratch_shapes=[
            pltpu.VMEM((2, D, H), jnp.bfloat16),
            pltpu.VMEM((S, 2, D, H), jnp.bfloat16),
            pltpu.VMEM((S, 2, D, H), jnp.bfloat16),
            pltpu.SemaphoreType.DMA((S,)),
            pltpu.SemaphoreType.DMA((S,)),
            pltpu.SemaphoreType.DMA((S,)),
            pltpu.SemaphoreType.DMA((S,)),
            pltpu.SemaphoreType.REGULAR,
            pltpu.SemaphoreType.REGULAR,
        ],
        compiler_params=pltpu.CompilerParams(collective_id=0),
    )(x, router_W, route_idx, expert_W)


# device time: 64797 ns/iter; 3.9610x vs baseline; 1.2643x over previous
import jax
import jax.numpy as jnp
from jax import lax
from jax.experimental import pallas as pl
from jax.experimental.pallas import tpu as pltpu

N_DEV = 16
N_EXP = 32
TOK = 512
D = 256
H = 512
S = 4
W = 2
R_HOPS = 8
L_HOPS = 7


def kernel(x, router_W, route_idx, expert_W):
    def body(x_ref, rw_ref, idx_ref, ew_ref, out_ref,
             ew_bf_ref, comm_r_ref, comm_l_ref,
             send_sems_r, recv_sems_r, send_sems_l, recv_sems_l,
             credit_r, credit_l):
        my = lax.axis_index("i")
        left = lax.rem(my + N_DEV - 1, N_DEV)
        right = lax.rem(my + 1, N_DEV)

        barrier_sem = pltpu.get_barrier_semaphore()
        for nbr in (left, right):
            pl.semaphore_signal(barrier_sem, inc=1, device_id=(nbr,),
                                device_id_type=pl.DeviceIdType.MESH)
        pl.semaphore_wait(barrier_sem, 2)

        pl.semaphore_signal(credit_r, inc=S, device_id=(left,),
                            device_id_type=pl.DeviceIdType.MESH)
        pl.semaphore_signal(credit_l, inc=S, device_id=(right,),
                            device_id_type=pl.DeviceIdType.MESH)

        ew_bf_ref[:, :, :] = ew_ref[:, :, :].astype(jnp.bfloat16)

        xv = x_ref[:, :]
        scores = jnp.dot(xv, rw_ref[:, :], preferred_element_type=jnp.float32)
        p = jnp.exp(scores - jnp.max(scores, axis=-1, keepdims=True))
        p = p / jnp.sum(p, axis=-1, keepdims=True)
        idx = idx_ref[:, :]
        idx0 = idx[:, 0:1]
        idx1 = idx[:, 1:2]
        eids = lax.broadcasted_iota(jnp.int32, (TOK, N_EXP), 1)
        p0 = jnp.sum(jnp.where(eids == idx0, p, 0.0), axis=-1, keepdims=True)
        p1 = jnp.sum(jnp.where(eids == idx1, p, 0.0), axis=-1, keepdims=True)
        w0 = p0 / (p0 + p1)
        w1 = p1 / (p0 + p1)

        def contrib(w_pair_bf, origin):
            e0 = 2 * origin
            e1 = e0 + 1
            g0 = jnp.where(idx0 == e0, w0, 0.0) + jnp.where(idx1 == e0, w1, 0.0)
            g1 = jnp.where(idx0 == e1, w0, 0.0) + jnp.where(idx1 == e1, w1, 0.0)
            xg = jnp.concatenate([xv * g0, xv * g1], axis=1)
            return jnp.dot(xg.astype(jnp.bfloat16), w_pair_bf.reshape(2 * D, H),
                           preferred_element_type=jnp.float32)

        def ring_desc(comm_ref, send_sems, recv_sems, h, w, nbr):
            return pltpu.make_async_remote_copy(
                src_ref=ew_bf_ref.at[w] if h == 0 else comm_ref.at[(h - 1) % S, w],
                dst_ref=comm_ref.at[h % S, w],
                send_sem=send_sems.at[h % S, w],
                recv_sem=recv_sems.at[h % S, w],
                device_id=(nbr,),
                device_id_type=pl.DeviceIdType.MESH,
            )

        def desc_r(h, w):
            return ring_desc(comm_r_ref, send_sems_r, recv_sems_r, h, w, right)

        def desc_l(h, w):
            return ring_desc(comm_l_ref, send_sems_l, recv_sems_l, h, w, left)

        for h in range(R_HOPS):
            pl.semaphore_wait(credit_r, 1)
            if h < L_HOPS:
                pl.semaphore_wait(credit_l, 1)
            for w in range(W):
                if h > 0:
                    desc_r(h - 1, w).wait_recv()
                desc_r(h, w).start()
                if h > 0:
                    desc_l(h - 1, w).wait_recv()
                if h < L_HOPS:
                    desc_l(h, w).start()

            if h == 0:
                out_ref[:, :] = contrib(ew_bf_ref[:, :, :], my)
            else:
                out_ref[:, :] += contrib(comm_r_ref[(h - 1) % S],
                                         lax.rem(my - h + N_DEV, N_DEV))
                out_ref[:, :] += contrib(comm_l_ref[(h - 1) % S],
                                         lax.rem(my + h, N_DEV))

            for w in range(W):
                desc_r(h, w).wait_send()
                if h < L_HOPS:
                    desc_l(h, w).wait_send()
            if 1 <= h <= R_HOPS - S:
                pl.semaphore_signal(credit_r, inc=1, device_id=(left,),
                                    device_id_type=pl.DeviceIdType.MESH)
            if 1 <= h <= L_HOPS - S:
                pl.semaphore_signal(credit_l, inc=1, device_id=(right,),
                                    device_id_type=pl.DeviceIdType.MESH)

        for w in range(W):
            desc_r(R_HOPS - 1, w).wait_recv()
        out_ref[:, :] += contrib(comm_r_ref[(R_HOPS - 1) % S],
                                 lax.rem(my - R_HOPS + N_DEV, N_DEV))

    return pl.pallas_call(
        body,
        out_shape=jax.ShapeDtypeStruct((TOK, H), jnp.float32),
        in_specs=[pl.BlockSpec(memory_space=pltpu.VMEM)] * 4,
        out_specs=pl.BlockSpec(memory_space=pltpu.VMEM),
        scratch_shapes=[
            pltpu.VMEM((2, D, H), jnp.bfloat16),
            pltpu.VMEM((S, 2, D, H), jnp.bfloat16),
            pltpu.VMEM((S, 2, D, H), jnp.bfloat16),
            pltpu.SemaphoreType.DMA((S, W)),
            pltpu.SemaphoreType.DMA((S, W)),
            pltpu.SemaphoreType.DMA((S, W)),
            pltpu.SemaphoreType.DMA((S, W)),
            pltpu.SemaphoreType.REGULAR,
            pltpu.SemaphoreType.REGULAR,
        ],
        compiler_params=pltpu.CompilerParams(collective_id=0),
    )(x, router_W, route_idx, expert_W)
